# baseline (speedup 1.0000x reference)
import dataclasses
import functools

import jax
import jax.numpy as jnp
from jax import lax
from jax.experimental import pallas as pl
from jax.experimental.pallas import tpu as pltpu
from jax.experimental.pallas import tpu_sc as plsc

B, L, V, E, H = 1024, 50, 1000000, 64, 64
G4 = 4 * H


def _lstm_body(e_ref, wih_ref, whh_ref, b_ref, out_ref, h_ref, c_ref):
    t = pl.program_id(0)

    @pl.when(t == 0)
    def _():
        h_ref[...] = jnp.zeros((H, B), jnp.float32)
        c_ref[...] = jnp.zeros((H, B), jnp.float32)

    h = h_ref[...].astype(jnp.bfloat16)
    c = c_ref[...]
    et = e_ref[...].astype(jnp.bfloat16)

    gates = (
        jnp.dot(wih_ref[...], et, preferred_element_type=jnp.float32)
        + jnp.dot(whh_ref[...], h, preferred_element_type=jnp.float32)
        + b_ref[...]
    )

    def sig(z):
        return 0.5 * jnp.tanh(0.5 * z) + 0.5

    i = sig(gates[0 * H : 1 * H])
    f = sig(gates[1 * H : 2 * H])
    g = jnp.tanh(gates[2 * H : 3 * H])
    o = sig(gates[3 * H : 4 * H])
    c = f * c + i * g
    h = o * jnp.tanh(c)
    h_ref[...] = h
    c_ref[...] = c
    out_ref[...] = h.reshape(1, H, B)


def _lstm_tc(e_t, wih, whh, bias2):
    return pl.pallas_call(
        _lstm_body,
        grid=(L,),
        in_specs=[
            pl.BlockSpec((E, B), lambda t: (0, t)),
            pl.BlockSpec((G4, E), lambda t: (0, 0)),
            pl.BlockSpec((G4, H), lambda t: (0, 0)),
            pl.BlockSpec((G4, 1), lambda t: (0, 0)),
        ],
        out_specs=pl.BlockSpec((1, H, B), lambda t: (t, 0, 0)),
        out_shape=jax.ShapeDtypeStruct((L, H, B), jnp.float32),
        scratch_shapes=[
            pltpu.VMEM((H, B), jnp.float32),
            pltpu.VMEM((H, B), jnp.float32),
        ],
    )(e_t, wih, whh, bias2)


def kernel(x, emb, W_ih, W_hh, b_ih, b_hh):
    eT = jnp.take(emb.T, x.T.reshape(-1), axis=1)
    bias2 = (b_ih + b_hh).reshape(G4, 1)
    wih16 = W_ih.astype(jnp.bfloat16)
    whh16 = W_hh.astype(jnp.bfloat16)
    o = _lstm_tc(eT, wih16, whh16, bias2)
    return o.transpose(2, 0, 1)


# speedup vs baseline: 1.0004x; 1.0004x over previous
import jax
import jax.numpy as jnp
from jax.experimental import pallas as pl
from jax.experimental.pallas import tpu as pltpu

B, L, V, E, H = 1024, 50, 1000000, 64, 64
G4 = 4 * H


def _lstm_body(e_ref, wih_ref, whh_ref, b_ref, out_ref, h_ref, c_ref):
    t = pl.program_id(0)

    @pl.when(t == 0)
    def _():
        h_ref[...] = jnp.zeros((H, B), jnp.float32)
        c_ref[...] = jnp.zeros((H, B), jnp.float32)

    h = h_ref[...].astype(jnp.bfloat16)
    c = c_ref[...]
    et = e_ref[...].astype(jnp.bfloat16)

    gates = (
        jnp.dot(wih_ref[...], et, preferred_element_type=jnp.float32)
        + jnp.dot(whh_ref[...], h, preferred_element_type=jnp.float32)
        + b_ref[...]
    )

    def sig(z):
        return 0.5 * jnp.tanh(0.5 * z) + 0.5

    i = sig(gates[0 * H : 1 * H])
    f = sig(gates[1 * H : 2 * H])
    g = jnp.tanh(gates[2 * H : 3 * H])
    o = sig(gates[3 * H : 4 * H])
    c = f * c + i * g
    h = o * jnp.tanh(c)
    h_ref[...] = h
    c_ref[...] = c
    out_ref[...] = h.reshape(1, H, B)


def _lstm_tc(e_t, wih, whh, bias2):
    return pl.pallas_call(
        _lstm_body,
        grid=(L,),
        in_specs=[
            pl.BlockSpec((E, B), lambda t: (0, t)),
            pl.BlockSpec((G4, E), lambda t: (0, 0)),
            pl.BlockSpec((G4, H), lambda t: (0, 0)),
            pl.BlockSpec((G4, 1), lambda t: (0, 0)),
        ],
        out_specs=pl.BlockSpec((1, H, B), lambda t: (t, 0, 0)),
        out_shape=jax.ShapeDtypeStruct((L, H, B), jnp.float32),
        scratch_shapes=[
            pltpu.VMEM((H, B), jnp.float32),
            pltpu.VMEM((H, B), jnp.float32),
        ],
    )(e_t, wih, whh, bias2)


def kernel(x, emb, W_ih, W_hh, b_ih, b_hh):
    eT = jnp.take(emb.T, x.T.reshape(-1), axis=1)
    bias2 = (b_ih + b_hh).reshape(G4, 1)
    wih16 = W_ih.astype(jnp.bfloat16)
    whh16 = W_hh.astype(jnp.bfloat16)
    o = _lstm_tc(eT, wih16, whh16, bias2)
    return o.transpose(2, 0, 1)
